# baseline (speedup 1.0000x reference)
import jax
import jax.numpy as jnp
from jax.experimental import pallas as pl
from jax.experimental.pallas import tpu as pltpu

WINDOW_SIZE = 16.0
N_ITER = 8
INV_SIGMA2 = 1.0 / (2.0 * WINDOW_SIZE * WINDOW_SIZE)
EPS = 1e-8
BATCH_PER_GROUP = 4


def _build_e(d, io, tri):
    T = io.shape[0]

    cum = jnp.dot(d, tri, preferred_element_type=jnp.float32)

    total = jnp.maximum(cum[:, T - 1 : T], 1.0)
    centers = (cum - 0.5 * d) * (jnp.float32(T) / total)

    diff = (io + 0.5) - centers
    logits = -(diff * diff) * INV_SIGMA2

    m = jnp.max(logits, axis=1, keepdims=True)
    return jnp.exp(logits - m)


def _stgw_body(ds_ref, xs_ref, out_ref, w_ref):
    T = xs_ref.shape[1]
    nb = BATCH_PER_GROUP
    k = pl.program_id(1)

    @pl.when(k == 0)
    def _build():
        ii = jax.lax.broadcasted_iota(jnp.int32, (T, T), 0)
        ij = jax.lax.broadcasted_iota(jnp.int32, (T, T), 1)
        tri = (ii <= ij).astype(jnp.float32)
        io = ii.astype(jnp.float32)

        es = [
            _build_e(ds_ref[i].astype(jnp.float32), io, tri) for i in range(nb)
        ]

        a_ = [jnp.sum(e, axis=1, keepdims=True) for e in es]
        b_ = [jnp.ones((1, T), dtype=jnp.float32) for _ in range(nb)]
        v_ = [None] * nb
        for _ in range(N_ITER):
            r_ = [1.0 / a for a in a_]
            s_ = [
                jnp.sum(es[i] * r_[i], axis=0, keepdims=True) for i in range(nb)
            ]
            b_ = [s_[i] + EPS * b_[i] for i in range(nb)]
            v_ = [1.0 / b for b in b_]
            z_ = [
                jnp.sum(es[i] * v_[i], axis=1, keepdims=True) for i in range(nb)
            ]
            a_ = [z_[i] + EPS * a_[i] for i in range(nb)]

        for i in range(nb):
            w_ref[i] = es[i] * v_[i] * (1.0 / a_[i])

    w = w_ref[pl.ds(k, 1)][0]
    out_ref[0] = jnp.dot(w, xs_ref[0], preferred_element_type=jnp.float32)


@jax.jit
def kernel(xs, ds):
    B, T, D = xs.shape
    ds3 = ds.reshape(B, 1, T)
    nb = BATCH_PER_GROUP
    return pl.pallas_call(
        _stgw_body,
        grid=(B // nb, nb),
        in_specs=[
            pl.BlockSpec((nb, 1, T), lambda g, k: (g, 0, 0)),
            pl.BlockSpec((1, T, D), lambda g, k: (g * nb + k, 0, 0)),
        ],
        out_specs=pl.BlockSpec((1, T, D), lambda g, k: (g * nb + k, 0, 0)),
        out_shape=jax.ShapeDtypeStruct((B, T, D), jnp.float32),
        scratch_shapes=[pltpu.VMEM((nb, T, T), jnp.float32)],
    )(ds3, xs)


# speedup vs baseline: 1.2515x; 1.2515x over previous
import jax
import jax.numpy as jnp
from jax.experimental import pallas as pl
from jax.experimental.pallas import tpu as pltpu

WINDOW_SIZE = 16.0
N_ITER = 8
INV_SIGMA2 = 1.0 / (2.0 * WINDOW_SIZE * WINDOW_SIZE)
EPS = 1e-8
BATCH_PER_STEP = 4


def _build_e(d, io, tri):
    T = io.shape[0]

    cum = jnp.dot(d, tri, preferred_element_type=jnp.float32)

    total = jnp.maximum(cum[:, T - 1 : T], 1.0)
    centers = (cum - 0.5 * d) * (jnp.float32(T) / total)

    diff = (io + 0.5) - centers
    logits = -(diff * diff) * INV_SIGMA2

    m = jnp.max(logits, axis=1, keepdims=True)
    return jnp.exp(logits - m)


def _stgw_body(ds_ref, xs_ref, out_ref):
    T = xs_ref.shape[1]
    ii = jax.lax.broadcasted_iota(jnp.int32, (T, T), 0)
    ij = jax.lax.broadcasted_iota(jnp.int32, (T, T), 1)
    tri = (ii <= ij).astype(jnp.float32)
    io = ii.astype(jnp.float32)

    nb = BATCH_PER_STEP
    es = [_build_e(ds_ref[k].astype(jnp.float32), io, tri) for k in range(nb)]

    a_ = [jnp.sum(e, axis=1, keepdims=True) for e in es]
    b_ = [jnp.ones((1, T), dtype=jnp.float32) for _ in range(nb)]
    v_ = [None] * nb
    for _ in range(N_ITER):
        r_ = [1.0 / a for a in a_]
        s_ = [jnp.sum(es[k] * r_[k], axis=0, keepdims=True) for k in range(nb)]
        b_ = [s_[k] + EPS * b_[k] for k in range(nb)]
        v_ = [1.0 / b for b in b_]
        z_ = [jnp.sum(es[k] * v_[k], axis=1, keepdims=True) for k in range(nb)]
        a_ = [z_[k] + EPS * a_[k] for k in range(nb)]

    for k in range(nb):
        ev = es[k] * v_[k]
        ys = jnp.dot(ev, xs_ref[k], preferred_element_type=jnp.float32)
        out_ref[k] = ys * (1.0 / a_[k])


@jax.jit
def kernel(xs, ds):
    B, T, D = xs.shape
    ds3 = ds.reshape(B, 1, T)
    bb = BATCH_PER_STEP
    return pl.pallas_call(
        _stgw_body,
        grid=(B // bb,),
        in_specs=[
            pl.BlockSpec((bb, 1, T), lambda b: (b, 0, 0)),
            pl.BlockSpec((bb, T, D), lambda b: (b, 0, 0)),
        ],
        out_specs=pl.BlockSpec((bb, T, D), lambda b: (b, 0, 0)),
        out_shape=jax.ShapeDtypeStruct((B, T, D), jnp.float32),
    )(ds3, xs)


# speedup vs baseline: 1.2584x; 1.0056x over previous
import jax
import jax.numpy as jnp
from jax.experimental import pallas as pl

WINDOW_SIZE = 16.0
N_ITER = 8
INV_SIGMA2 = 1.0 / (2.0 * WINDOW_SIZE * WINDOW_SIZE)
EPS = 1e-8
BATCH_PER_STEP = 4


def _build_e(d, io, tri):
    T = io.shape[0]

    cum = jnp.dot(d, tri, preferred_element_type=jnp.float32)

    total = jnp.maximum(cum[:, T - 1 : T], 1.0)
    centers = (cum - 0.5 * d) * (jnp.float32(T) / total)

    diff = (io + 0.5) - centers
    logits = -(diff * diff) * INV_SIGMA2

    m = jnp.max(logits, axis=1, keepdims=True)
    return jnp.exp(logits - m)


def _stgw_body(ds_ref, xs_ref, out_ref):
    T = xs_ref.shape[1]
    ii = jax.lax.broadcasted_iota(jnp.int32, (T, T), 0)
    ij = jax.lax.broadcasted_iota(jnp.int32, (T, T), 1)
    tri = (ii <= ij).astype(jnp.float32)
    io = ii.astype(jnp.float32)

    nb = BATCH_PER_STEP
    es = [_build_e(ds_ref[k].astype(jnp.float32), io, tri) for k in range(nb)]

    a_ = [jnp.sum(e, axis=1, keepdims=True) for e in es]
    b_ = [jnp.ones((1, T), dtype=jnp.float32) for _ in range(nb)]
    v_ = [None] * nb
    for _ in range(N_ITER):
        r_ = [1.0 / a for a in a_]
        s_ = [jnp.sum(es[k] * r_[k], axis=0, keepdims=True) for k in range(nb)]
        b_ = [s_[k] + EPS * b_[k] for k in range(nb)]
        v_ = [1.0 / b for b in b_]
        z_ = [jnp.sum(es[k] * v_[k], axis=1, keepdims=True) for k in range(nb)]
        a_ = [z_[k] + EPS * a_[k] for k in range(nb)]

    for k in range(nb):
        ev = es[k] * v_[k]
        ys = jnp.dot(ev, xs_ref[k], preferred_element_type=jnp.float32)
        out_ref[k] = ys * (1.0 / a_[k])


@jax.jit
def kernel(xs, ds):
    B, T, D = xs.shape
    ds3 = ds.reshape(B, 1, T)
    bb = BATCH_PER_STEP
    return pl.pallas_call(
        _stgw_body,
        grid=(B // bb,),
        in_specs=[
            pl.BlockSpec((bb, 1, T), lambda b: (b, 0, 0)),
            pl.BlockSpec((bb, T, D), lambda b: (b, 0, 0)),
        ],
        out_specs=pl.BlockSpec((bb, T, D), lambda b: (b, 0, 0)),
        out_shape=jax.ShapeDtypeStruct((B, T, D), jnp.float32),
    )(ds3, xs)
